# baseline (speedup 1.0000x reference)
import functools

import jax
import jax.numpy as jnp
from jax import lax
from jax.experimental import pallas as pl
from jax.experimental.pallas import tpu as pltpu
from jax.experimental.pallas import tpu_sc as plsc

NC = 2
NS = 16
NW = NC * NS
EB = 128
M_PAD = 10112
RPT = M_PAD // NS
BN = 1000


@functools.cache
def _make_prop(D, nb):
    mesh = plsc.VectorSubcoreMesh(core_axis_name="c", subcore_axis_name="s")

    def body(xs_hbm, edges_hbm, zeros_hbm, out_hbm, idx_v, rows_v, acc, sem):
        c = lax.axis_index("c")
        s = lax.axis_index("s")
        wid = s * NC + c
        r0 = pl.multiple_of(s * RPT, 8)
        pltpu.sync_copy(zeros_hbm.at[pl.ds(r0, RPT)], acc.at[pl.ds(r0, RPT)])
        plsc.subcore_barrier()

        def step(j, carry):
            pltpu.sync_copy(edges_hbm.at[wid, j], idx_v)
            pltpu.async_copy(xs_hbm.at[idx_v.at[0]], rows_v, sem).wait()
            pltpu.sync_copy(rows_v, acc.at[idx_v.at[1]], add=True)
            return carry

        lax.fori_loop(0, nb, step, 0)
        plsc.subcore_barrier()
        pltpu.sync_copy(acc.at[pl.ds(r0, RPT)], out_hbm.at[c, pl.ds(r0, RPT)])

    return pl.kernel(
        body,
        out_type=jax.ShapeDtypeStruct((NC, M_PAD, D), jnp.float32),
        mesh=mesh,
        compiler_params=pltpu.CompilerParams(use_tc_tiling_on_sc=False),
        scratch_types=[
            pltpu.VMEM((2, EB), jnp.int32),
            pltpu.VMEM((EB, D), jnp.float32),
            pltpu.VMEM_SHARED((M_PAD, D), jnp.float32),
            pltpu.SemaphoreType.DMA,
        ],
        name=f"sc_prop_d{D}_nb{nb}",
    )


def _pack_edges(src, dst):
    e = src.shape[0]
    nb = -(-e // (NW * EB))
    tot = NW * nb * EB
    srcp = jnp.zeros((tot,), jnp.int32).at[:e].set(src.astype(jnp.int32))
    junk = 10000 + (jnp.arange(tot, dtype=jnp.int32) % (M_PAD - 10000))
    dstp = junk.at[:e].set(dst.astype(jnp.int32))
    packed = jnp.stack(
        [srcp.reshape(NW, nb, EB), dstp.reshape(NW, nb, EB)], axis=2)
    return packed, nb


def _rspec(w):
    return pl.BlockSpec((BN, w), lambda i: (i, 0))


def _pspecs(d):
    return [pl.BlockSpec((1, BN, d), lambda i, p=p: (p, i, 0)) for p in (0, 1)]


def _fspec(shape):
    return pl.BlockSpec(shape, lambda i: tuple(0 for _ in shape))


def _out(n, w):
    return jax.ShapeDtypeStruct((n, w), jnp.float32)


def _dinv_from_deg(dpair):
    def body(p0, p1, o):
        d = p0[0][:, 0:1] + p1[0][:, 0:1]
        o[...] = jnp.where(d > 0, lax.rsqrt(d), 0.0)

    n = BN * 10
    return pl.pallas_call(
        body, grid=(n // BN,),
        in_specs=_pspecs(64), out_specs=_rspec(1), out_shape=_out(n, 1),
        name="tc_dinv")(dpair, dpair)


def _rowscale(x, dinv):
    def body(x_r, s_r, o):
        o[...] = x_r[...] * s_r[...]

    n, w = x.shape
    return pl.pallas_call(
        body, grid=(n // BN,),
        in_specs=[_rspec(w), _rspec(1)], out_specs=_rspec(w),
        out_shape=_out(n, w), name="tc_rowscale")(x, dinv)


def _combine_pair(qpair, dinv):
    def body(q0, q1, s_r, x_o, xs_o):
        x = (q0[0] + q1[0]) * 0.125
        x_o[...] = x
        xs_o[...] = x * s_r[...]

    n = BN * 10
    return pl.pallas_call(
        body, grid=(n // BN,),
        in_specs=_pspecs(64) + [_rspec(1)],
        out_specs=[_rspec(64), _rspec(64)],
        out_shape=[_out(n, 64), _out(n, 64)],
        name="tc_combine_pair")(qpair, qpair, dinv)


def _mm(a, b):
    return jnp.dot(a, b, preferred_element_type=jnp.float32)


def _gates_t0(x, ppair, dinv, w, head, n_div):
    dx = x.shape[1]

    def body(x_r, p0, p1, s_r, W0, W1, B, *rest):
        if head == "u":
            Wu, bu, h_o, hs_o, u_o = rest
        elif head == "elem":
            Ws, bs, Wrf, brf, h_o, hs_o, s1_o, s16_o, rf_o = rest
        else:
            h_o, hs_o = rest
        nd = -s_r[...]
        px = (p0[0] + p1[0]) * nd
        g = _mm(x_r[...], W0[...]) + _mm(px, W1[...]) + B[...]
        Z = jax.nn.sigmoid(g[:, 0:64])
        Ht = jnp.tanh(g[:, 128:192])
        Hn = (1.0 - Z) * Ht
        h_o[...] = Hn
        hs_o[...] = Hn * s_r[...]
        if head == "u":
            u_o[...] = _mm(Hn, Wu[...]) + bu[...]
        elif head == "elem":
            s1 = _mm(Hn, Ws[...]) + bs[...]
            s1_o[...] = s1
            s16_o[...] = jnp.broadcast_to(s1, (BN, 64))

            @pl.when(pl.program_id(0) == 0)
            def _():
                rf_o[...] = brf[...]

            rf_o[...] += _mm(jnp.sum(Hn, axis=0, keepdims=True) * (1.0 / n_div),
                             Wrf[...])

    n = BN * 10
    ins = [_rspec(dx)] + _pspecs(dx) + [_rspec(1), _fspec((dx, 192)),
                                        _fspec((dx, 192)), _fspec((1, 192))]
    args = [x, ppair, ppair, dinv, w["Wx0"], w["Wx1"], w["bsum"]]
    outs = [_rspec(64), _rspec(64)]
    oshp = [_out(n, 64), _out(n, 64)]
    if head == "u":
        ins += [_fspec((64, 3)), _fspec((1, 3))]
        args += [w["Wu"], w["bu"]]
        outs += [_rspec(3)]
        oshp += [_out(n, 3)]
    elif head == "elem":
        ins += [_fspec((64, 1)), _fspec((1, 1)), _fspec((64, 1)),
                _fspec((1, 1))]
        args += [w["Ws"], w["bs"], w["Wrf"], w["brf"]]
        outs += [_rspec(1), _rspec(64), _fspec((1, 1))]
        oshp += [_out(n, 1), _out(n, 64), _out(1, 1)]
    return pl.pallas_call(
        body, grid=(n // BN,), in_specs=ins, out_specs=outs, out_shape=oshp,
        name=f"tc_gates_t0_{head}_{dx}")(*args)


def _gates1(x, pxpair, H, phpair, dinv, w):
    dx = x.shape[1]

    def body(x_r, px0, px1, h_r, ph0, ph1, s_r, Wx0, Wx1, Bx, Wh0, Wh1, Bh,
             z_o, ah_o, hr_o, hrs_o):
        nd = -s_r[...]
        px = (px0[0] + px1[0]) * nd
        gx = _mm(x_r[...], Wx0[...]) + _mm(px, Wx1[...]) + Bx[...]
        ph = (ph0[0] + ph1[0]) * nd
        gh = _mm(h_r[...], Wh0[...]) + _mm(ph, Wh1[...]) + Bh[...]
        z_o[...] = jax.nn.sigmoid(gx[:, 0:64] + gh[:, 0:64])
        r = jax.nn.sigmoid(gx[:, 64:128] + gh[:, 64:128])
        ah_o[...] = gx[:, 128:192]
        hr = h_r[...] * r
        hr_o[...] = hr
        hrs_o[...] = hr * s_r[...]

    n = BN * 10
    ins = ([_rspec(dx)] + _pspecs(dx) + [_rspec(64)] + _pspecs(64) +
           [_rspec(1), _fspec((dx, 192)), _fspec((dx, 192)), _fspec((1, 192)),
            _fspec((64, 128)), _fspec((64, 128)), _fspec((1, 128))])
    args = [x, pxpair, pxpair, H, phpair, phpair, dinv,
            w["Wx0"], w["Wx1"], w["bx"], w["Wh0"], w["Wh1"], w["bh"]]
    return pl.pallas_call(
        body, grid=(n // BN,), in_specs=ins,
        out_specs=[_rspec(64)] * 4,
        out_shape=[_out(n, 64)] * 4,
        name=f"tc_gates1_{dx}")(*args)


def _gates2(Z, AH, HR, prpair, H, dinv, w, head, n_div):
    def body(z_r, ah_r, hr_r, pr0, pr1, h_r, s_r, Wh0, Wh1, Bh, *rest):
        if head == "u":
            Wu, bu, h_o, hs_o, u_o = rest
        elif head == "elem":
            Ws, bs, Wrf, brf, h_o, hs_o, s1_o, s16_o, rf_o = rest
        else:
            h_o, hs_o = rest
        pr = (pr0[0] + pr1[0]) * (-s_r[...])
        ht = jnp.tanh(ah_r[...] + _mm(hr_r[...], Wh0[...]) +
                      _mm(pr, Wh1[...]) + Bh[...])
        z = z_r[...]
        hn = z * h_r[...] + (1.0 - z) * ht
        h_o[...] = hn
        hs_o[...] = hn * s_r[...]
        if head == "u":
            u_o[...] = _mm(hn, Wu[...]) + bu[...]
        elif head == "elem":
            s1 = _mm(hn, Ws[...]) + bs[...]
            s1_o[...] = s1
            s16_o[...] = jnp.broadcast_to(s1, (BN, 64))

            @pl.when(pl.program_id(0) == 0)
            def _():
                rf_o[...] = brf[...]

            rf_o[...] += _mm(jnp.sum(hn, axis=0, keepdims=True) * (1.0 / n_div),
                             Wrf[...])

    n = BN * 10
    ins = ([_rspec(64)] * 3 + _pspecs(64) + [_rspec(64), _rspec(1),
           _fspec((64, 64)), _fspec((64, 64)), _fspec((1, 64))])
    args = [Z, AH, HR, prpair, prpair, H, dinv, w["Whh0"], w["Whh1"], w["bhh"]]
    outs = [_rspec(64), _rspec(64)]
    oshp = [_out(n, 64), _out(n, 64)]
    if head == "u":
        ins += [_fspec((64, 3)), _fspec((1, 3))]
        args += [w["Wu"], w["bu"]]
        outs += [_rspec(3)]
        oshp += [_out(n, 3)]
    elif head == "elem":
        ins += [_fspec((64, 1)), _fspec((1, 1)), _fspec((64, 1)),
                _fspec((1, 1))]
        args += [w["Ws"], w["bs"], w["Wrf"], w["brf"]]
        outs += [_rspec(1), _rspec(64), _fspec((1, 1))]
        oshp += [_out(n, 1), _out(n, 64), _out(1, 1)]
    return pl.pallas_call(
        body, grid=(n // BN,), in_specs=ins, out_specs=outs, out_shape=oshp,
        name=f"tc_gates2_{head}")(*args)


def _build_x(xn_raw, x_prev, u, apair, cpair, dinv):
    def body(xn_r, xp_r, u_r, a0, a1, c0, c1, s_r, x_o, xs_o):
        s_node = (a0[0][:, 0:1] + a1[0][:, 0:1]) / (
            c0[0][:, 0:1] + c1[0][:, 0:1] + 1e-06)
        v = u_r[...] - xp_r[:, 3:6]
        x = jnp.concatenate(
            [xn_r[:, 0:3], u_r[...], xn_r[:, 6:8], v, s_node,
             jnp.zeros((BN, 52), jnp.float32)], axis=1)
        x_o[...] = x
        xs_o[...] = x * s_r[...]

    n = BN * 10
    ins = ([_rspec(64), _rspec(64), _rspec(3)] + _pspecs(64) + _pspecs(64) +
           [_rspec(1)])
    args = [xn_raw, x_prev, u, apair, apair, cpair, cpair, dinv]
    return pl.pallas_call(
        body, grid=(n // BN,), in_specs=ins,
        out_specs=[_rspec(64), _rspec(64)],
        out_shape=[_out(n, 64), _out(n, 64)],
        name="tc_build_x")(*args)


def _prep_gru(p, dx_pad):
    def catw(names, k, pad):
        w = jnp.concatenate([p[nm]["W"][k] for nm in names], axis=1)
        return jnp.pad(w, ((0, pad - w.shape[0]), (0, 0)))

    def catb(names):
        return jnp.concatenate([p[nm]["b"] for nm in names]).reshape(1, -1)

    xg = ["xz", "xr", "xh"]
    hg = ["hz", "hr"]
    bx = catb(xg)
    return {
        "Wx0": catw(xg, 0, dx_pad), "Wx1": catw(xg, 1, dx_pad), "bx": bx,
        "Wh0": catw(hg, 0, 64), "Wh1": catw(hg, 1, 64), "bh": catb(hg),
        "Whh0": p["hh"]["W"][0], "Whh1": p["hh"]["W"][1],
        "bhh": p["hh"]["b"].reshape(1, 64),
        "bsum": bx + catb(["hz", "hr", "hh"]),
    }


def kernel(X_node_seq, node_edge_index, elem_edge_index, elem_nodes_idx,
           params):
    Tn, N, _ = X_node_seq.shape
    Ne = elem_nodes_idx.shape[0]
    f32 = jnp.float32

    Xp = jnp.pad(X_node_seq, ((0, 0), (0, 0), (0, 52)))
    src_n = node_edge_index[0]
    src_e = elem_edge_index[0]
    eni_flat = elem_nodes_idx.reshape(-1)
    elem_of_slot = jnp.repeat(jnp.arange(Ne, dtype=jnp.int32), 8)
    node_of_slot = eni_flat.astype(jnp.int32)
    zeros_e = jnp.zeros_like(src_n)
    EN, nb_n = _pack_edges(src_n, node_edge_index[1])
    EE, nb_e = _pack_edges(src_e, elem_edge_index[1])
    EG, nb_g = _pack_edges(node_of_slot, elem_of_slot)
    ES, _ = _pack_edges(elem_of_slot, node_of_slot)
    EDN, _ = _pack_edges(zeros_e, src_n)
    EDE, _ = _pack_edges(jnp.zeros_like(src_e), src_e)
    ECNT, _ = _pack_edges(jnp.zeros_like(elem_of_slot), node_of_slot)
    ones8 = jnp.ones((8, 64), f32)
    z64 = jnp.zeros((M_PAD, 64), f32)

    prop64_n = _make_prop(64, nb_n)
    prop64_e = _make_prop(64, nb_e)
    prop64_g = _make_prop(64, nb_g)

    wn1 = _prep_gru(params["node_gru1"], 64)
    wn2 = _prep_gru(params["node_gru2"], 64)
    we1 = _prep_gru(params["elem_gru1"], 64)
    we2 = _prep_gru(params["elem_gru2"], 64)
    wn2 = dict(wn2, Wu=params["head_u"]["W"],
               bu=params["head_u"]["b"].reshape(1, 3))
    we2 = dict(we2, Ws=params["head_s_elem"]["W"],
               bs=params["head_s_elem"]["b"].reshape(1, 1),
               Wrf=params["head_rf2"]["W"],
               brf=params["head_rf2"]["b"].reshape(1, 1))

    dinv_n = _dinv_from_deg(prop64_n(ones8, EDN, z64))
    dinv_e = _dinv_from_deg(prop64_e(ones8, EDE, z64))
    cnt_pair = prop64_g(ones8, ECNT, z64)

    us, ss, rfs = [], [], []

    x_cur = Xp[0]
    x_s = _rowscale(x_cur, dinv_n)
    p = prop64_n(x_s, EN, z64)
    Hn1, Hn1s = _gates_t0(x_cur, p, dinv_n, wn1, "plain", N)
    pn2x = prop64_n(Hn1s, EN, z64)
    Hn2, Hn2s, u_cur = _gates_t0(Hn1, pn2x, dinv_n, wn2, "u", N)
    q = prop64_g(Hn2, EG, z64)
    xe, xes = _combine_pair(q, dinv_e)
    p = prop64_e(xes, EE, z64)
    He1, He1s = _gates_t0(xe, p, dinv_e, we1, "plain", Ne)
    pe2x = prop64_e(He1s, EE, z64)
    He2, He2s, s1, s16v, rf = _gates_t0(He1, pe2x, dinv_e, we2, "elem", Ne)
    us.append(u_cur)
    ss.append(s1[:, 0])
    rfs.append(rf[0, 0])

    for t in range(1, Tn):
        ap = prop64_g(s16v, ES, z64)
        xt, xts = _build_x(Xp[t], x_cur, u_cur, ap, cnt_pair, dinv_n)
        x_cur = xt
        px = prop64_n(xts, EN, z64)
        Z, AH, HR, HRs = _gates1(xt, px, Hn1, pn2x, dinv_n, wn1)
        pr = prop64_n(HRs, EN, z64)
        Hn1, Hn1s = _gates2(Z, AH, HR, pr, Hn1, dinv_n, wn1, "plain", N)
        px2 = prop64_n(Hn1s, EN, z64)
        ph2 = prop64_n(Hn2s, EN, z64)
        Z, AH, HR, HRs = _gates1(Hn1, px2, Hn2, ph2, dinv_n, wn2)
        pr = prop64_n(HRs, EN, z64)
        Hn2, Hn2s, u_cur = _gates2(Z, AH, HR, pr, Hn2, dinv_n, wn2, "u", N)
        pn2x = px2
        q = prop64_g(Hn2, EG, z64)
        xe, xes = _combine_pair(q, dinv_e)
        px = prop64_e(xes, EE, z64)
        Z, AH, HR, HRs = _gates1(xe, px, He1, pe2x, dinv_e, we1)
        pr = prop64_e(HRs, EE, z64)
        He1, He1s = _gates2(Z, AH, HR, pr, He1, dinv_e, we1, "plain", Ne)
        px2 = prop64_e(He1s, EE, z64)
        ph2 = prop64_e(He2s, EE, z64)
        Z, AH, HR, HRs = _gates1(He1, px2, He2, ph2, dinv_e, we2)
        pr = prop64_e(HRs, EE, z64)
        He2, He2s, s1, s16v, rf = _gates2(Z, AH, HR, pr, He2, dinv_e, we2,
                                          "elem", Ne)
        pe2x = px2
        us.append(u_cur)
        ss.append(s1[:, 0])
        rfs.append(rf[0, 0])

    return (jnp.stack(us), jnp.stack(ss), jnp.stack(rfs))


# speedup vs baseline: 2.2482x; 2.2482x over previous
import functools

import jax
import jax.numpy as jnp
from jax import lax
from jax.experimental import pallas as pl
from jax.experimental.pallas import tpu as pltpu
from jax.experimental.pallas import tpu_sc as plsc

NC = 2
NS = 16
NW = NC * NS
EB = 128
M_PAD = 10112
RPT = M_PAD // NS
BN = 1000


@functools.cache
def _make_prop(D, nb):
    mesh = plsc.VectorSubcoreMesh(core_axis_name="c", subcore_axis_name="s")

    def body(xs_hbm, edges_hbm, zeros_hbm, out_hbm, idx_v, rows_v, acc, sem):
        c = lax.axis_index("c")
        s = lax.axis_index("s")
        wid = s * NC + c
        r0 = pl.multiple_of(s * RPT, 8)
        pltpu.sync_copy(zeros_hbm.at[pl.ds(r0, RPT)], acc.at[pl.ds(r0, RPT)])
        plsc.subcore_barrier()

        def step(j, carry):
            pltpu.sync_copy(edges_hbm.at[wid, j], idx_v)
            pltpu.async_copy(xs_hbm.at[idx_v.at[0]], rows_v, sem).wait()
            pltpu.sync_copy(rows_v, acc.at[idx_v.at[1]], add=True)
            return carry

        lax.fori_loop(0, nb, step, 0)
        plsc.subcore_barrier()
        pltpu.sync_copy(acc.at[pl.ds(r0, RPT)], out_hbm.at[c, pl.ds(r0, RPT)])

    return pl.kernel(
        body,
        out_type=jax.ShapeDtypeStruct((NC, M_PAD, D), jnp.float32),
        mesh=mesh,
        compiler_params=pltpu.CompilerParams(use_tc_tiling_on_sc=False),
        scratch_types=[
            pltpu.VMEM((2, EB), jnp.int32),
            pltpu.VMEM((EB, D), jnp.float32),
            pltpu.VMEM_SHARED((M_PAD, D), jnp.float32),
            pltpu.SemaphoreType.DMA,
        ],
        name=f"sc_prop_d{D}_nb{nb}",
    )


def _pack_edges(src, dst):
    e = src.shape[0]
    nb = -(-e // (NW * EB))
    tot = NW * nb * EB
    srcp = jnp.zeros((tot,), jnp.int32).at[:e].set(src.astype(jnp.int32))
    junk = 10000 + (jnp.arange(tot, dtype=jnp.int32) % (M_PAD - 10000))
    dstp = junk.at[:e].set(dst.astype(jnp.int32))
    packed = jnp.stack(
        [srcp.reshape(NW, nb, EB), dstp.reshape(NW, nb, EB)], axis=2)
    return packed, nb


def _rspec(w):
    return pl.BlockSpec((BN, w), lambda i: (i, 0))


def _pspecs(d):
    return [pl.BlockSpec((1, BN, d), lambda i, p=p: (p, i, 0)) for p in (0, 1)]


def _fspec(shape):
    return pl.BlockSpec(shape, lambda i: tuple(0 for _ in shape))


def _out(n, w):
    return jax.ShapeDtypeStruct((n, w), jnp.float32)


def _dinv_from_deg(dpair):
    def body(p0, p1, o):
        d = p0[0][:, 0:1] + p1[0][:, 0:1]
        o[...] = jnp.where(d > 0, 1.0 / jnp.sqrt(d), 0.0)

    n = BN * 10
    return pl.pallas_call(
        body, grid=(n // BN,),
        in_specs=_pspecs(64), out_specs=_rspec(1), out_shape=_out(n, 1),
        name="tc_dinv")(dpair, dpair)


def _rowscale(x, dinv):
    def body(x_r, s_r, o):
        o[...] = x_r[...] * s_r[...]

    n, w = x.shape
    return pl.pallas_call(
        body, grid=(n // BN,),
        in_specs=[_rspec(w), _rspec(1)], out_specs=_rspec(w),
        out_shape=_out(n, w), name="tc_rowscale")(x, dinv)


def _combine_pair(qpair, dinv):
    def body(q0, q1, s_r, x_o, xs_o):
        x = (q0[0] + q1[0]) * 0.125
        x_o[...] = x
        xs_o[...] = x * s_r[...]

    n = BN * 10
    return pl.pallas_call(
        body, grid=(n // BN,),
        in_specs=_pspecs(64) + [_rspec(1)],
        out_specs=[_rspec(64), _rspec(64)],
        out_shape=[_out(n, 64), _out(n, 64)],
        name="tc_combine_pair")(qpair, qpair, dinv)


def _mm(a, b):
    return jnp.dot(a, b, preferred_element_type=jnp.float32)


def _gates_t0(x, ppair, dinv, w, head, n_div):
    dx = x.shape[1]

    def body(x_r, p0, p1, s_r, W0, W1, B, *rest):
        if head == "u":
            Wu, bu, h_o, hs_o, u_o = rest
        elif head == "elem":
            Ws, bs, Wrf, brf, h_o, hs_o, s1_o, s16_o, rf_o = rest
        else:
            h_o, hs_o = rest
        nd = -s_r[...]
        px = (p0[0] + p1[0]) * nd
        g = _mm(x_r[...], W0[...]) + _mm(px, W1[...]) + B[...]
        Z = jax.nn.sigmoid(g[:, 0:64])
        Ht = jnp.tanh(g[:, 128:192])
        Hn = (1.0 - Z) * Ht
        h_o[...] = Hn
        hs_o[...] = Hn * s_r[...]
        if head == "u":
            u_o[...] = _mm(Hn, Wu[...]) + bu[...]
        elif head == "elem":
            s1 = _mm(Hn, Ws[...]) + bs[...]
            s1_o[...] = s1
            s16_o[...] = jnp.broadcast_to(s1, (BN, 64))

            @pl.when(pl.program_id(0) == 0)
            def _():
                rf_o[...] = brf[...]

            rf_o[...] += _mm(jnp.sum(Hn, axis=0, keepdims=True) * (1.0 / n_div),
                             Wrf[...])

    n = BN * 10
    ins = [_rspec(dx)] + _pspecs(dx) + [_rspec(1), _fspec((dx, 192)),
                                        _fspec((dx, 192)), _fspec((1, 192))]
    args = [x, ppair, ppair, dinv, w["Wx0"], w["Wx1"], w["bsum"]]
    outs = [_rspec(64), _rspec(64)]
    oshp = [_out(n, 64), _out(n, 64)]
    if head == "u":
        ins += [_fspec((64, 3)), _fspec((1, 3))]
        args += [w["Wu"], w["bu"]]
        outs += [_rspec(3)]
        oshp += [_out(n, 3)]
    elif head == "elem":
        ins += [_fspec((64, 1)), _fspec((1, 1)), _fspec((64, 1)),
                _fspec((1, 1))]
        args += [w["Ws"], w["bs"], w["Wrf"], w["brf"]]
        outs += [_rspec(1), _rspec(64), _fspec((1, 1))]
        oshp += [_out(n, 1), _out(n, 64), _out(1, 1)]
    return pl.pallas_call(
        body, grid=(n // BN,), in_specs=ins, out_specs=outs, out_shape=oshp,
        name=f"tc_gates_t0_{head}_{dx}")(*args)


def _gates1(x, pxpair, H, phpair, dinv, w):
    dx = x.shape[1]

    def body(x_r, px0, px1, h_r, ph0, ph1, s_r, Wx0, Wx1, Bx, Wh0, Wh1, Bh,
             z_o, ah_o, hr_o, hrs_o):
        nd = -s_r[...]
        px = (px0[0] + px1[0]) * nd
        gx = _mm(x_r[...], Wx0[...]) + _mm(px, Wx1[...]) + Bx[...]
        ph = (ph0[0] + ph1[0]) * nd
        gh = _mm(h_r[...], Wh0[...]) + _mm(ph, Wh1[...]) + Bh[...]
        z_o[...] = jax.nn.sigmoid(gx[:, 0:64] + gh[:, 0:64])
        r = jax.nn.sigmoid(gx[:, 64:128] + gh[:, 64:128])
        ah_o[...] = gx[:, 128:192]
        hr = h_r[...] * r
        hr_o[...] = hr
        hrs_o[...] = hr * s_r[...]

    n = BN * 10
    ins = ([_rspec(dx)] + _pspecs(dx) + [_rspec(64)] + _pspecs(64) +
           [_rspec(1), _fspec((dx, 192)), _fspec((dx, 192)), _fspec((1, 192)),
            _fspec((64, 128)), _fspec((64, 128)), _fspec((1, 128))])
    args = [x, pxpair, pxpair, H, phpair, phpair, dinv,
            w["Wx0"], w["Wx1"], w["bx"], w["Wh0"], w["Wh1"], w["bh"]]
    return pl.pallas_call(
        body, grid=(n // BN,), in_specs=ins,
        out_specs=[_rspec(64)] * 4,
        out_shape=[_out(n, 64)] * 4,
        name=f"tc_gates1_{dx}")(*args)


def _gates2(Z, AH, HR, prpair, H, dinv, w, head, n_div):
    def body(z_r, ah_r, hr_r, pr0, pr1, h_r, s_r, Wh0, Wh1, Bh, *rest):
        if head == "u":
            Wu, bu, h_o, hs_o, u_o = rest
        elif head == "elem":
            Ws, bs, Wrf, brf, h_o, hs_o, s1_o, s16_o, rf_o = rest
        else:
            h_o, hs_o = rest
        pr = (pr0[0] + pr1[0]) * (-s_r[...])
        ht = jnp.tanh(ah_r[...] + _mm(hr_r[...], Wh0[...]) +
                      _mm(pr, Wh1[...]) + Bh[...])
        z = z_r[...]
        hn = z * h_r[...] + (1.0 - z) * ht
        h_o[...] = hn
        hs_o[...] = hn * s_r[...]
        if head == "u":
            u_o[...] = _mm(hn, Wu[...]) + bu[...]
        elif head == "elem":
            s1 = _mm(hn, Ws[...]) + bs[...]
            s1_o[...] = s1
            s16_o[...] = jnp.broadcast_to(s1, (BN, 64))

            @pl.when(pl.program_id(0) == 0)
            def _():
                rf_o[...] = brf[...]

            rf_o[...] += _mm(jnp.sum(hn, axis=0, keepdims=True) * (1.0 / n_div),
                             Wrf[...])

    n = BN * 10
    ins = ([_rspec(64)] * 3 + _pspecs(64) + [_rspec(64), _rspec(1),
           _fspec((64, 64)), _fspec((64, 64)), _fspec((1, 64))])
    args = [Z, AH, HR, prpair, prpair, H, dinv, w["Whh0"], w["Whh1"], w["bhh"]]
    outs = [_rspec(64), _rspec(64)]
    oshp = [_out(n, 64), _out(n, 64)]
    if head == "u":
        ins += [_fspec((64, 3)), _fspec((1, 3))]
        args += [w["Wu"], w["bu"]]
        outs += [_rspec(3)]
        oshp += [_out(n, 3)]
    elif head == "elem":
        ins += [_fspec((64, 1)), _fspec((1, 1)), _fspec((64, 1)),
                _fspec((1, 1))]
        args += [w["Ws"], w["bs"], w["Wrf"], w["brf"]]
        outs += [_rspec(1), _rspec(64), _fspec((1, 1))]
        oshp += [_out(n, 1), _out(n, 64), _out(1, 1)]
    return pl.pallas_call(
        body, grid=(n // BN,), in_specs=ins, out_specs=outs, out_shape=oshp,
        name=f"tc_gates2_{head}")(*args)


def _build_x(xn_raw, x_prev, u, apair, cpair, dinv):
    def body(xn_r, xp_r, u_r, a0, a1, c0, c1, s_r, x_o, xs_o):
        s_node = (a0[0][:, 0:1] + a1[0][:, 0:1]) / (
            c0[0][:, 0:1] + c1[0][:, 0:1] + 1e-06)
        v = u_r[...] - xp_r[:, 3:6]
        x = jnp.concatenate(
            [xn_r[:, 0:3], u_r[...], xn_r[:, 6:8], v, s_node,
             jnp.zeros((BN, 52), jnp.float32)], axis=1)
        x_o[...] = x
        xs_o[...] = x * s_r[...]

    n = BN * 10
    ins = ([_rspec(64), _rspec(64), _rspec(3)] + _pspecs(64) + _pspecs(64) +
           [_rspec(1)])
    args = [xn_raw, x_prev, u, apair, apair, cpair, cpair, dinv]
    return pl.pallas_call(
        body, grid=(n // BN,), in_specs=ins,
        out_specs=[_rspec(64), _rspec(64)],
        out_shape=[_out(n, 64), _out(n, 64)],
        name="tc_build_x")(*args)


def _prep_gru(p, dx_pad):
    def catw(names, k, pad):
        w = jnp.concatenate([p[nm]["W"][k] for nm in names], axis=1)
        return jnp.pad(w, ((0, pad - w.shape[0]), (0, 0)))

    def catb(names):
        return jnp.concatenate([p[nm]["b"] for nm in names]).reshape(1, -1)

    xg = ["xz", "xr", "xh"]
    hg = ["hz", "hr"]
    bx = catb(xg)
    return {
        "Wx0": catw(xg, 0, dx_pad), "Wx1": catw(xg, 1, dx_pad), "bx": bx,
        "Wh0": catw(hg, 0, 64), "Wh1": catw(hg, 1, 64), "bh": catb(hg),
        "Whh0": p["hh"]["W"][0], "Whh1": p["hh"]["W"][1],
        "bhh": p["hh"]["b"].reshape(1, 64),
        "bsum": bx + catb(["hz", "hr", "hh"]),
    }


def kernel(X_node_seq, node_edge_index, elem_edge_index, elem_nodes_idx,
           params):
    Tn, N, _ = X_node_seq.shape
    Ne = elem_nodes_idx.shape[0]
    f32 = jnp.float32

    Xp = jnp.pad(X_node_seq, ((0, 0), (0, 0), (0, 52)))
    src_n = node_edge_index[0]
    src_e = elem_edge_index[0]
    eni_flat = elem_nodes_idx.reshape(-1)
    elem_of_slot = jnp.repeat(jnp.arange(Ne, dtype=jnp.int32), 8)
    node_of_slot = eni_flat.astype(jnp.int32)
    EN, nb_n = _pack_edges(src_n, node_edge_index[1])
    EE, nb_e = _pack_edges(src_e, elem_edge_index[1])
    EG, nb_g = _pack_edges(node_of_slot, elem_of_slot)
    ES, _ = _pack_edges(elem_of_slot, node_of_slot)
    EDN, _ = _pack_edges(src_n, src_n)
    EDE, _ = _pack_edges(src_e, src_e)
    ECNT, _ = _pack_edges(node_of_slot, node_of_slot)
    ones_t = jnp.ones((M_PAD, 64), f32)
    z64 = jnp.zeros((M_PAD, 64), f32)

    prop64_n = _make_prop(64, nb_n)
    prop64_e = _make_prop(64, nb_e)
    prop64_g = _make_prop(64, nb_g)

    wn1 = _prep_gru(params["node_gru1"], 64)
    wn2 = _prep_gru(params["node_gru2"], 64)
    we1 = _prep_gru(params["elem_gru1"], 64)
    we2 = _prep_gru(params["elem_gru2"], 64)
    wn2 = dict(wn2, Wu=params["head_u"]["W"],
               bu=params["head_u"]["b"].reshape(1, 3))
    we2 = dict(we2, Ws=params["head_s_elem"]["W"],
               bs=params["head_s_elem"]["b"].reshape(1, 1),
               Wrf=params["head_rf2"]["W"],
               brf=params["head_rf2"]["b"].reshape(1, 1))

    dinv_n = _dinv_from_deg(prop64_n(ones_t, EDN, z64))
    dinv_e = _dinv_from_deg(prop64_e(ones_t, EDE, z64))
    cnt_pair = prop64_g(ones_t, ECNT, z64)

    us, ss, rfs = [], [], []

    x_cur = Xp[0]
    x_s = _rowscale(x_cur, dinv_n)
    p = prop64_n(x_s, EN, z64)
    Hn1, Hn1s = _gates_t0(x_cur, p, dinv_n, wn1, "plain", N)
    pn2x = prop64_n(Hn1s, EN, z64)
    Hn2, Hn2s, u_cur = _gates_t0(Hn1, pn2x, dinv_n, wn2, "u", N)
    q = prop64_g(Hn2, EG, z64)
    xe, xes = _combine_pair(q, dinv_e)
    p = prop64_e(xes, EE, z64)
    He1, He1s = _gates_t0(xe, p, dinv_e, we1, "plain", Ne)
    pe2x = prop64_e(He1s, EE, z64)
    He2, He2s, s1, s16v, rf = _gates_t0(He1, pe2x, dinv_e, we2, "elem", Ne)
    us.append(u_cur)
    ss.append(s1[:, 0])
    rfs.append(rf[0, 0])

    for t in range(1, Tn):
        ap = prop64_g(s16v, ES, z64)
        xt, xts = _build_x(Xp[t], x_cur, u_cur, ap, cnt_pair, dinv_n)
        x_cur = xt
        px = prop64_n(xts, EN, z64)
        Z, AH, HR, HRs = _gates1(xt, px, Hn1, pn2x, dinv_n, wn1)
        pr = prop64_n(HRs, EN, z64)
        Hn1, Hn1s = _gates2(Z, AH, HR, pr, Hn1, dinv_n, wn1, "plain", N)
        px2 = prop64_n(Hn1s, EN, z64)
        ph2 = prop64_n(Hn2s, EN, z64)
        Z, AH, HR, HRs = _gates1(Hn1, px2, Hn2, ph2, dinv_n, wn2)
        pr = prop64_n(HRs, EN, z64)
        Hn2, Hn2s, u_cur = _gates2(Z, AH, HR, pr, Hn2, dinv_n, wn2, "u", N)
        pn2x = px2
        q = prop64_g(Hn2, EG, z64)
        xe, xes = _combine_pair(q, dinv_e)
        px = prop64_e(xes, EE, z64)
        Z, AH, HR, HRs = _gates1(xe, px, He1, pe2x, dinv_e, we1)
        pr = prop64_e(HRs, EE, z64)
        He1, He1s = _gates2(Z, AH, HR, pr, He1, dinv_e, we1, "plain", Ne)
        px2 = prop64_e(He1s, EE, z64)
        ph2 = prop64_e(He2s, EE, z64)
        Z, AH, HR, HRs = _gates1(He1, px2, He2, ph2, dinv_e, we2)
        pr = prop64_e(HRs, EE, z64)
        He2, He2s, s1, s16v, rf = _gates2(Z, AH, HR, pr, He2, dinv_e, we2,
                                          "elem", Ne)
        pe2x = px2
        us.append(u_cur)
        ss.append(s1[:, 0])
        rfs.append(rf[0, 0])

    return (jnp.stack(us), jnp.stack(ss), jnp.stack(rfs))


# speedup vs baseline: 6.4493x; 2.8687x over previous
import functools

import jax
import jax.numpy as jnp
from jax import lax
from jax.experimental import pallas as pl
from jax.experimental.pallas import tpu as pltpu
from jax.experimental.pallas import tpu_sc as plsc

NC = 2
NS = 16
NW = NC * NS
EB = 128
K = 4
M_PAD = 10112
RPT = M_PAD // NS
BN = 1000


@functools.cache
def _make_prop(D, nbs):
    mesh = plsc.VectorSubcoreMesh(core_axis_name="c", subcore_axis_name="s")

    def body(xs_hbm, edges_hbm, zeros_hbm, out_hbm,
             idx0, idx1, rows0, rows1, acc, sem0, sem1):
        c = lax.axis_index("c")
        s = lax.axis_index("s")
        wid = s * NC + c
        r0 = pl.multiple_of(s * RPT, 8)
        pltpu.sync_copy(zeros_hbm.at[pl.ds(r0, RPT)], acc.at[pl.ds(r0, RPT)])
        plsc.subcore_barrier()

        def fire(idx_b, rows_b, sem_b, sb):
            pltpu.sync_copy(edges_hbm.at[wid, sb], idx_b)
            for k in range(K):
                pltpu.async_copy(xs_hbm.at[idx_b.at[0, k]],
                                 rows_b.at[pl.ds(k * EB, EB)], sem_b)

        def drain_scatter(idx_b, rows_b, sem_b):
            pltpu.make_async_copy(zeros_hbm.at[pl.ds(0, K * EB)], rows_b,
                                  sem_b).wait()
            for k in range(K):
                pltpu.sync_copy(rows_b.at[pl.ds(k * EB, EB)],
                                acc.at[idx_b.at[1, k]], add=True)

        fire(idx0, rows0, sem0, 0)

        def pair(i2, carry):
            sb0 = 2 * i2
            fire(idx1, rows1, sem1, sb0 + 1)
            drain_scatter(idx0, rows0, sem0)

            @pl.when(sb0 + 2 < nbs)
            def _():
                fire(idx0, rows0, sem0, sb0 + 2)

            drain_scatter(idx1, rows1, sem1)
            return carry

        lax.fori_loop(0, nbs // 2, pair, 0)
        plsc.subcore_barrier()
        pltpu.sync_copy(acc.at[pl.ds(r0, RPT)], out_hbm.at[c, pl.ds(r0, RPT)])

    return pl.kernel(
        body,
        out_type=jax.ShapeDtypeStruct((NC, M_PAD, D), jnp.float32),
        mesh=mesh,
        compiler_params=pltpu.CompilerParams(use_tc_tiling_on_sc=False),
        scratch_types=[
            pltpu.VMEM((2, K, EB), jnp.int32),
            pltpu.VMEM((2, K, EB), jnp.int32),
            pltpu.VMEM((K * EB, D), jnp.float32),
            pltpu.VMEM((K * EB, D), jnp.float32),
            pltpu.VMEM_SHARED((M_PAD, D), jnp.float32),
            pltpu.SemaphoreType.DMA,
            pltpu.SemaphoreType.DMA,
        ],
        name=f"sc_prop_d{D}_nbs{nbs}",
    )


def _pack_edges(src, dst):
    e = src.shape[0]
    nb = -(-e // (NW * EB))
    nb = -(-nb // (2 * K)) * (2 * K)
    nbs = nb // K
    tot = NW * nb * EB
    ar = jnp.arange(tot, dtype=jnp.int32)
    srcp = (ar % 10000).at[:e].set(src.astype(jnp.int32))
    junk = 10000 + (ar % (M_PAD - 10000))
    dstp = junk.at[:e].set(dst.astype(jnp.int32))
    packed = jnp.stack(
        [srcp.reshape(NW, nbs, K, EB), dstp.reshape(NW, nbs, K, EB)], axis=2)
    return packed, nbs


def _rspec(w):
    return pl.BlockSpec((BN, w), lambda i: (i, 0))


def _pspecs(d):
    return [pl.BlockSpec((1, BN, d), lambda i, p=p: (p, i, 0)) for p in (0, 1)]


def _fspec(shape):
    return pl.BlockSpec(shape, lambda i: tuple(0 for _ in shape))


def _out(n, w):
    return jax.ShapeDtypeStruct((n, w), jnp.float32)


def _dinv_from_deg(dpair):
    def body(p0, p1, o):
        d = p0[0][:, 0:1] + p1[0][:, 0:1]
        o[...] = jnp.where(d > 0, 1.0 / jnp.sqrt(d), 0.0)

    n = BN * 10
    return pl.pallas_call(
        body, grid=(n // BN,),
        in_specs=_pspecs(64), out_specs=_rspec(1), out_shape=_out(n, 1),
        name="tc_dinv")(dpair, dpair)


def _rowscale(x, dinv):
    def body(x_r, s_r, o):
        o[...] = x_r[...] * s_r[...]

    n, w = x.shape
    return pl.pallas_call(
        body, grid=(n // BN,),
        in_specs=[_rspec(w), _rspec(1)], out_specs=_rspec(w),
        out_shape=_out(n, w), name="tc_rowscale")(x, dinv)


def _combine_pair(qpair, dinv):
    def body(q0, q1, s_r, x_o, xs_o):
        x = (q0[0] + q1[0]) * 0.125
        x_o[...] = x
        xs_o[...] = x * s_r[...]

    n = BN * 10
    return pl.pallas_call(
        body, grid=(n // BN,),
        in_specs=_pspecs(64) + [_rspec(1)],
        out_specs=[_rspec(64), _rspec(64)],
        out_shape=[_out(n, 64), _out(n, 64)],
        name="tc_combine_pair")(qpair, qpair, dinv)


def _mm(a, b):
    return jnp.dot(a, b, preferred_element_type=jnp.float32)


def _gates_t0(x, ppair, dinv, w, head, n_div):
    dx = x.shape[1]

    def body(x_r, p0, p1, s_r, W0, W1, B, *rest):
        if head == "u":
            Wu, bu, h_o, hs_o, u_o = rest
        elif head == "elem":
            Ws, bs, Wrf, brf, h_o, hs_o, s1_o, s16_o, rf_o, colacc = rest
        else:
            h_o, hs_o = rest
        nd = -s_r[...]
        px = (p0[0] + p1[0]) * nd
        g = _mm(x_r[...], W0[...]) + _mm(px, W1[...]) + B[...]
        Z = jax.nn.sigmoid(g[:, 0:64])
        Ht = jnp.tanh(g[:, 128:192])
        Hn = (1.0 - Z) * Ht
        h_o[...] = Hn
        hs_o[...] = Hn * s_r[...]
        if head == "u":
            u_o[...] = _mm(Hn, Wu[...]) + bu[...]
        elif head == "elem":
            s1 = _mm(Hn, Ws[...]) + bs[...]
            s1_o[...] = s1
            s16_o[...] = jnp.broadcast_to(s1, (BN, 64))

            @pl.when(pl.program_id(0) == 0)
            def _():
                colacc[...] = jnp.zeros_like(colacc)

            colacc[...] += jnp.sum(Hn, axis=0, keepdims=True)

            @pl.when(pl.program_id(0) == pl.num_programs(0) - 1)
            def _():
                rf_o[...] = jnp.dot(
                    colacc[...] / n_div, Wrf[...],
                    preferred_element_type=jnp.float32,
                    precision=lax.Precision.HIGHEST) + brf[...]

    n = BN * 10
    ins = [_rspec(dx)] + _pspecs(dx) + [_rspec(1), _fspec((dx, 192)),
                                        _fspec((dx, 192)), _fspec((1, 192))]
    args = [x, ppair, ppair, dinv, w["Wx0"], w["Wx1"], w["bsum"]]
    outs = [_rspec(64), _rspec(64)]
    oshp = [_out(n, 64), _out(n, 64)]
    if head == "u":
        ins += [_fspec((64, 3)), _fspec((1, 3))]
        args += [w["Wu"], w["bu"]]
        outs += [_rspec(3)]
        oshp += [_out(n, 3)]
    elif head == "elem":
        ins += [_fspec((64, 1)), _fspec((1, 1)), _fspec((64, 1)),
                _fspec((1, 1))]
        args += [w["Ws"], w["bs"], w["Wrf"], w["brf"]]
        outs += [_rspec(1), _rspec(64), _fspec((1, 1))]
        oshp += [_out(n, 1), _out(n, 64), _out(1, 1)]
    scr = [pltpu.VMEM((1, 64), jnp.float32)] if head == "elem" else []
    return pl.pallas_call(
        body, grid=(n // BN,), in_specs=ins, out_specs=outs, out_shape=oshp,
        scratch_shapes=scr,
        name=f"tc_gates_t0_{head}_{dx}")(*args)


def _gates1(x, pxpair, H, phpair, dinv, w):
    dx = x.shape[1]

    def body(x_r, px0, px1, h_r, ph0, ph1, s_r, Wx0, Wx1, Bx, Wh0, Wh1, Bh,
             z_o, ah_o, hr_o, hrs_o):
        nd = -s_r[...]
        px = (px0[0] + px1[0]) * nd
        gx = _mm(x_r[...], Wx0[...]) + _mm(px, Wx1[...]) + Bx[...]
        ph = (ph0[0] + ph1[0]) * nd
        gh = _mm(h_r[...], Wh0[...]) + _mm(ph, Wh1[...]) + Bh[...]
        z_o[...] = jax.nn.sigmoid(gx[:, 0:64] + gh[:, 0:64])
        r = jax.nn.sigmoid(gx[:, 64:128] + gh[:, 64:128])
        ah_o[...] = gx[:, 128:192]
        hr = h_r[...] * r
        hr_o[...] = hr
        hrs_o[...] = hr * s_r[...]

    n = BN * 10
    ins = ([_rspec(dx)] + _pspecs(dx) + [_rspec(64)] + _pspecs(64) +
           [_rspec(1), _fspec((dx, 192)), _fspec((dx, 192)), _fspec((1, 192)),
            _fspec((64, 128)), _fspec((64, 128)), _fspec((1, 128))])
    args = [x, pxpair, pxpair, H, phpair, phpair, dinv,
            w["Wx0"], w["Wx1"], w["bx"], w["Wh0"], w["Wh1"], w["bh"]]
    return pl.pallas_call(
        body, grid=(n // BN,), in_specs=ins,
        out_specs=[_rspec(64)] * 4,
        out_shape=[_out(n, 64)] * 4,
        name=f"tc_gates1_{dx}")(*args)


def _gates2(Z, AH, HR, prpair, H, dinv, w, head, n_div):
    def body(z_r, ah_r, hr_r, pr0, pr1, h_r, s_r, Wh0, Wh1, Bh, *rest):
        if head == "u":
            Wu, bu, h_o, hs_o, u_o = rest
        elif head == "elem":
            Ws, bs, Wrf, brf, h_o, hs_o, s1_o, s16_o, rf_o, colacc = rest
        else:
            h_o, hs_o = rest
        pr = (pr0[0] + pr1[0]) * (-s_r[...])
        ht = jnp.tanh(ah_r[...] + _mm(hr_r[...], Wh0[...]) +
                      _mm(pr, Wh1[...]) + Bh[...])
        z = z_r[...]
        hn = z * h_r[...] + (1.0 - z) * ht
        h_o[...] = hn
        hs_o[...] = hn * s_r[...]
        if head == "u":
            u_o[...] = _mm(hn, Wu[...]) + bu[...]
        elif head == "elem":
            s1 = _mm(hn, Ws[...]) + bs[...]
            s1_o[...] = s1
            s16_o[...] = jnp.broadcast_to(s1, (BN, 64))

            @pl.when(pl.program_id(0) == 0)
            def _():
                colacc[...] = jnp.zeros_like(colacc)

            colacc[...] += jnp.sum(hn, axis=0, keepdims=True)

            @pl.when(pl.program_id(0) == pl.num_programs(0) - 1)
            def _():
                rf_o[...] = jnp.dot(
                    colacc[...] / n_div, Wrf[...],
                    preferred_element_type=jnp.float32,
                    precision=lax.Precision.HIGHEST) + brf[...]

    n = BN * 10
    ins = ([_rspec(64)] * 3 + _pspecs(64) + [_rspec(64), _rspec(1),
           _fspec((64, 64)), _fspec((64, 64)), _fspec((1, 64))])
    args = [Z, AH, HR, prpair, prpair, H, dinv, w["Whh0"], w["Whh1"], w["bhh"]]
    outs = [_rspec(64), _rspec(64)]
    oshp = [_out(n, 64), _out(n, 64)]
    if head == "u":
        ins += [_fspec((64, 3)), _fspec((1, 3))]
        args += [w["Wu"], w["bu"]]
        outs += [_rspec(3)]
        oshp += [_out(n, 3)]
    elif head == "elem":
        ins += [_fspec((64, 1)), _fspec((1, 1)), _fspec((64, 1)),
                _fspec((1, 1))]
        args += [w["Ws"], w["bs"], w["Wrf"], w["brf"]]
        outs += [_rspec(1), _rspec(64), _fspec((1, 1))]
        oshp += [_out(n, 1), _out(n, 64), _out(1, 1)]
    scr = [pltpu.VMEM((1, 64), jnp.float32)] if head == "elem" else []
    return pl.pallas_call(
        body, grid=(n // BN,), in_specs=ins, out_specs=outs, out_shape=oshp,
        scratch_shapes=scr,
        name=f"tc_gates2_{head}")(*args)


def _build_x(xn_raw, x_prev, u, apair, cpair, dinv):
    def body(xn_r, xp_r, u_r, a0, a1, c0, c1, s_r, x_o, xs_o):
        s_node = (a0[0][:, 0:1] + a1[0][:, 0:1]) / (
            c0[0][:, 0:1] + c1[0][:, 0:1] + 1e-06)
        v = u_r[...] - xp_r[:, 3:6]
        x = jnp.concatenate(
            [xn_r[:, 0:3], u_r[...], xn_r[:, 6:8], v, s_node,
             jnp.zeros((BN, 52), jnp.float32)], axis=1)
        x_o[...] = x
        xs_o[...] = x * s_r[...]

    n = BN * 10
    ins = ([_rspec(64), _rspec(64), _rspec(3)] + _pspecs(64) + _pspecs(64) +
           [_rspec(1)])
    args = [xn_raw, x_prev, u, apair, apair, cpair, cpair, dinv]
    return pl.pallas_call(
        body, grid=(n // BN,), in_specs=ins,
        out_specs=[_rspec(64), _rspec(64)],
        out_shape=[_out(n, 64), _out(n, 64)],
        name="tc_build_x")(*args)


def _prep_gru(p, dx_pad):
    def catw(names, k, pad):
        w = jnp.concatenate([p[nm]["W"][k] for nm in names], axis=1)
        return jnp.pad(w, ((0, pad - w.shape[0]), (0, 0)))

    def catb(names):
        return jnp.concatenate([p[nm]["b"] for nm in names]).reshape(1, -1)

    xg = ["xz", "xr", "xh"]
    hg = ["hz", "hr"]
    bx = catb(xg)
    return {
        "Wx0": catw(xg, 0, dx_pad), "Wx1": catw(xg, 1, dx_pad), "bx": bx,
        "Wh0": catw(hg, 0, 64), "Wh1": catw(hg, 1, 64), "bh": catb(hg),
        "Whh0": p["hh"]["W"][0], "Whh1": p["hh"]["W"][1],
        "bhh": p["hh"]["b"].reshape(1, 64),
        "bsum": bx + catb(["hz", "hr", "hh"]),
    }


def kernel(X_node_seq, node_edge_index, elem_edge_index, elem_nodes_idx,
           params):
    Tn, N, _ = X_node_seq.shape
    Ne = elem_nodes_idx.shape[0]
    f32 = jnp.float32

    Xp = jnp.pad(X_node_seq, ((0, 0), (0, 0), (0, 52)))
    src_n = node_edge_index[0]
    src_e = elem_edge_index[0]
    eni_flat = elem_nodes_idx.reshape(-1)
    elem_of_slot = jnp.repeat(jnp.arange(Ne, dtype=jnp.int32), 8)
    node_of_slot = eni_flat.astype(jnp.int32)
    EN, nb_n = _pack_edges(src_n, node_edge_index[1])
    EE, nb_e = _pack_edges(src_e, elem_edge_index[1])
    EG, nb_g = _pack_edges(node_of_slot, elem_of_slot)
    ES, _ = _pack_edges(elem_of_slot, node_of_slot)
    EDN, _ = _pack_edges(src_n, src_n)
    EDE, _ = _pack_edges(src_e, src_e)
    ECNT, _ = _pack_edges(node_of_slot, node_of_slot)
    ones_t = jnp.ones((M_PAD, 64), f32)
    z64 = jnp.zeros((M_PAD, 64), f32)

    prop64_n = _make_prop(64, nb_n)
    prop64_e = _make_prop(64, nb_e)
    prop64_g = _make_prop(64, nb_g)

    wn1 = _prep_gru(params["node_gru1"], 64)
    wn2 = _prep_gru(params["node_gru2"], 64)
    we1 = _prep_gru(params["elem_gru1"], 64)
    we2 = _prep_gru(params["elem_gru2"], 64)
    wn2 = dict(wn2, Wu=params["head_u"]["W"],
               bu=params["head_u"]["b"].reshape(1, 3))
    we2 = dict(we2, Ws=params["head_s_elem"]["W"],
               bs=params["head_s_elem"]["b"].reshape(1, 1),
               Wrf=params["head_rf2"]["W"],
               brf=params["head_rf2"]["b"].reshape(1, 1))

    dinv_n = _dinv_from_deg(prop64_n(ones_t, EDN, z64))
    dinv_e = _dinv_from_deg(prop64_e(ones_t, EDE, z64))
    cnt_pair = prop64_g(ones_t, ECNT, z64)

    us, ss, rfs = [], [], []

    x_cur = Xp[0]
    x_s = _rowscale(x_cur, dinv_n)
    p = prop64_n(x_s, EN, z64)
    Hn1, Hn1s = _gates_t0(x_cur, p, dinv_n, wn1, "plain", N)
    pn2x = prop64_n(Hn1s, EN, z64)
    Hn2, Hn2s, u_cur = _gates_t0(Hn1, pn2x, dinv_n, wn2, "u", N)
    q = prop64_g(Hn2, EG, z64)
    xe, xes = _combine_pair(q, dinv_e)
    p = prop64_e(xes, EE, z64)
    He1, He1s = _gates_t0(xe, p, dinv_e, we1, "plain", Ne)
    pe2x = prop64_e(He1s, EE, z64)
    He2, He2s, s1, s16v, rf = _gates_t0(He1, pe2x, dinv_e, we2, "elem", Ne)
    us.append(u_cur)
    ss.append(s1[:, 0])
    rfs.append(rf[0, 0])

    for t in range(1, Tn):
        ap = prop64_g(s16v, ES, z64)
        xt, xts = _build_x(Xp[t], x_cur, u_cur, ap, cnt_pair, dinv_n)
        x_cur = xt
        px = prop64_n(xts, EN, z64)
        Z, AH, HR, HRs = _gates1(xt, px, Hn1, pn2x, dinv_n, wn1)
        pr = prop64_n(HRs, EN, z64)
        Hn1, Hn1s = _gates2(Z, AH, HR, pr, Hn1, dinv_n, wn1, "plain", N)
        px2 = prop64_n(Hn1s, EN, z64)
        ph2 = prop64_n(Hn2s, EN, z64)
        Z, AH, HR, HRs = _gates1(Hn1, px2, Hn2, ph2, dinv_n, wn2)
        pr = prop64_n(HRs, EN, z64)
        Hn2, Hn2s, u_cur = _gates2(Z, AH, HR, pr, Hn2, dinv_n, wn2, "u", N)
        pn2x = px2
        q = prop64_g(Hn2, EG, z64)
        xe, xes = _combine_pair(q, dinv_e)
        px = prop64_e(xes, EE, z64)
        Z, AH, HR, HRs = _gates1(xe, px, He1, pe2x, dinv_e, we1)
        pr = prop64_e(HRs, EE, z64)
        He1, He1s = _gates2(Z, AH, HR, pr, He1, dinv_e, we1, "plain", Ne)
        px2 = prop64_e(He1s, EE, z64)
        ph2 = prop64_e(He2s, EE, z64)
        Z, AH, HR, HRs = _gates1(He1, px2, He2, ph2, dinv_e, we2)
        pr = prop64_e(HRs, EE, z64)
        He2, He2s, s1, s16v, rf = _gates2(Z, AH, HR, pr, He2, dinv_e, we2,
                                          "elem", Ne)
        pe2x = px2
        us.append(u_cur)
        ss.append(s1[:, 0])
        rfs.append(rf[0, 0])

    return (jnp.stack(us), jnp.stack(ss), jnp.stack(rfs))
